# baseline (speedup 1.0000x reference)
import functools

import jax
import jax.numpy as jnp
from jax import lax
from jax.experimental import pallas as pl
from jax.experimental.pallas import tpu as pltpu
from jax.experimental.pallas import tpu_sc as plsc

B, S, H = 1024, 200, 768
LANES = 16
NVREG = H // LANES
HW = H // 2
CHUNK = 32
NB_B = 4
NB_S = 8
BBLK = B // NB_B
SBLK = S // NB_S
EPS = 1e-12


def _rsqrt_vec(v):
    i = lax.bitcast_convert_type(v, jnp.int32)
    i = jnp.int32(0x5F3759DF) - (i >> 1)
    y = lax.bitcast_convert_type(i, jnp.float32)
    half = v * 0.5
    for _ in range(3):
        y = y * (1.5 - half * y * y)
    return y


def _pack_table(x):
    xb = x.astype(jnp.bfloat16)
    u = lax.bitcast_convert_type(xb, jnp.uint16).astype(jnp.uint32)
    packed = (u[:, :HW] << 16) | u[:, HW:]
    return lax.bitcast_convert_type(packed, jnp.int32)


def _build_kernel(num_cores, num_subcores):
    nw = num_cores * num_subcores
    tokens = B * S
    per_w = tokens // nw
    n_chunks = per_w // CHUNK
    mesh = plsc.VectorSubcoreMesh(core_axis_name="c", subcore_axis_name="s")

    @functools.partial(
        pl.kernel,
        mesh=mesh,
        out_type=jax.ShapeDtypeStruct((tokens, H), jnp.float32),
        compiler_params=pltpu.CompilerParams(needs_layout_passes=False,
                                             use_tc_tiling_on_sc=False),
        scratch_types=(
            [pltpu.VMEM((CHUNK,), jnp.int32) for _ in range(2)]
            + [pltpu.VMEM((CHUNK,), jnp.int32) for _ in range(2)]
            + [pltpu.VMEM((CHUNK, HW), jnp.int32) for _ in range(2)]
            + [pltpu.VMEM((CHUNK, HW), jnp.int32) for _ in range(2)]
            + [pltpu.VMEM((SBLK, HW), jnp.int32)]
            + [pltpu.VMEM((CHUNK, H), jnp.float32) for _ in range(2)]
            + [pltpu.VMEM((H,), jnp.float32) for _ in range(2)]
            + [pltpu.SemaphoreType.DMA for _ in range(10)]
        ),
    )
    def emb_kernel(ids_hbm, tids_hbm, tok_hbm, pos_hbm, typ_hbm, gamma_hbm,
                   beta_hbm, out_hbm,
                   idtok0, idtok1, idtyp0, idtyp1, tokb0, tokb1, typb0, typb1,
                   pos_pin, ob0, ob1, g_v, b_v,
                   s_gt0, s_gt1, s_gy0, s_gy1,
                   s_it0, s_it1, s_iy0, s_iy1, s_o0, s_o1):
        idtok = (idtok0, idtok1)
        idtyp = (idtyp0, idtyp1)
        tokb = (tokb0, tokb1)
        typb = (typb0, typb1)
        ob = (ob0, ob1)
        s_gt = (s_gt0, s_gt1)
        s_gy = (s_gy0, s_gy1)
        s_it = (s_it0, s_it1)
        s_iy = (s_iy0, s_iy1)
        s_o = (s_o0, s_o1)

        wid = lax.axis_index("s") * num_cores + lax.axis_index("c")
        bb = lax.rem(wid, NB_B)
        sb = wid // NB_B
        pltpu.sync_copy(gamma_hbm, g_v)
        pltpu.sync_copy(beta_hbm, b_v)
        pltpu.sync_copy(pos_hbm.at[pl.ds(sb * SBLK, SBLK)], pos_pin)
        row_iota = jnp.arange(LANES, dtype=jnp.int32)

        def chunk_coords(k):
            s_local = lax.rem(k, SBLK)
            b0 = bb * BBLK + CHUNK * (k // SBLK)
            s = sb * SBLK + s_local
            return s_local, b0, s

        def issue_ids(k, p):
            _, b0, s = chunk_coords(k)
            base = s * B + b0
            pltpu.async_copy(ids_hbm.at[pl.ds(base, CHUNK)], idtok[p], s_it[p])
            pltpu.async_copy(tids_hbm.at[pl.ds(base, CHUNK)], idtyp[p], s_iy[p])

        def wait_ids(p):
            pltpu.make_async_copy(ids_hbm.at[pl.ds(0, CHUNK)], idtok[p],
                                  s_it[p]).wait()
            pltpu.make_async_copy(tids_hbm.at[pl.ds(0, CHUNK)], idtyp[p],
                                  s_iy[p]).wait()

        def issue_gathers(k, p):
            pltpu.async_copy(tok_hbm.at[idtok[p]], tokb[p], s_gt[p])
            pltpu.async_copy(typ_hbm.at[idtyp[p]], typb[p], s_gy[p])

        def wait_gathers(p):
            pltpu.make_async_copy(tok_hbm.at[idtok[p]], tokb[p], s_gt[p]).wait()
            pltpu.make_async_copy(typ_hbm.at[idtyp[p]], typb[p], s_gy[p]).wait()

        def issue_out(k, p):
            _, b0, s = chunk_coords(k)
            o1 = (b0 + row_iota) * S + s
            o2 = o1 + LANES * S
            pltpu.async_copy(ob[p].at[pl.ds(0, LANES)], out_hbm.at[o1], s_o[p])
            pltpu.async_copy(ob[p].at[pl.ds(LANES, LANES)], out_hbm.at[o2],
                             s_o[p])

        def wait_out(p):
            pltpu.make_async_copy(ob[p].at[pl.ds(0, LANES)],
                                  out_hbm.at[row_iota], s_o[p]).wait()
            pltpu.make_async_copy(ob[p].at[pl.ds(LANES, LANES)],
                                  out_hbm.at[row_iota], s_o[p]).wait()

        bcast_dnums = lax.GatherDimensionNumbers(
            offset_dims=(), collapsed_slice_dims=(0,), start_index_map=(0,))

        def bcast(vec, lane):
            idx = jnp.full((LANES, 1), lane, jnp.int32)
            return lax.gather(vec, idx, dimension_numbers=bcast_dnums,
                              slice_sizes=(1,),
                              mode=lax.GatherScatterMode.PROMISE_IN_BOUNDS)

        def unpack(w):
            hi = lax.bitcast_convert_type(w, jnp.float32)
            lo = lax.bitcast_convert_type(w << 16, jnp.float32)
            return hi, lo

        def compute_group(p, g, s_lv):
            tb, yb, o = tokb[p], typb[p], ob[p]
            rows = row_iota + g * LANES
            nacc = 2
            ph = 4

            def pass1_body(carry, wrap):
                accs = list(carry[:4 * nacc])
                hvs = list(carry[4 * nacc:])
                tws = [plsc.load_gather(tb, [rows, hvs[u]]) for u in range(ph)]
                yws = [plsc.load_gather(yb, [rows, hvs[u]]) for u in range(ph)]
                pws = [plsc.load_gather(pos_pin, [s_lv, hvs[u]])
                       for u in range(ph)]
                for u in range(ph):
                    thi, tlo = unpack(tws[u])
                    yhi, ylo = unpack(yws[u])
                    phi, plo = unpack(pws[u])
                    chi = (thi + yhi) + phi
                    clo = (tlo + ylo) + plo
                    plsc.store_scatter(o, [rows, hvs[u]], chi)
                    plsc.store_scatter(o, [rows, hvs[u] + HW], clo)
                    a = u % nacc
                    accs[a] = accs[a] + chi
                    accs[nacc + a] = accs[nacc + a] + clo
                    accs[2 * nacc + a] = accs[2 * nacc + a] + chi * chi
                    accs[3 * nacc + a] = accs[3 * nacc + a] + clo * clo
                nxt = []
                for u in range(ph):
                    hv = hvs[u] + ph
                    if wrap:
                        hv = jnp.where(hv >= HW, hv - HW, hv)
                    nxt.append(hv)
                return tuple(accs) + tuple(nxt)

            zero = jnp.zeros((LANES,), jnp.float32)
            hv0 = [row_iota + u for u in range(ph)]
            n_safe = (HW - LANES - ph) // ph
            carry = lax.fori_loop(0, n_safe,
                                  lambda blk, c: pass1_body(c, False),
                                  (zero,) * (4 * nacc) + tuple(hv0))
            carry = lax.fori_loop(n_safe, HW // ph,
                                  lambda blk, c: pass1_body(c, True),
                                  carry)
            s1 = (carry[0] + carry[1]) + (carry[2] + carry[3])
            s2 = (carry[4] + carry[5]) + (carry[6] + carry[7])
            mv = s1 * (1.0 / H)
            var = jnp.maximum(s2 * (1.0 / H) - mv * mv, 0.0)
            rv = _rsqrt_vec(var + EPS)
            mrv = mv * rv

            th = 8
            for t0 in (g * LANES, g * LANES + th):
                rvs = [bcast(rv, (t0 % LANES) + t) for t in range(th)]
                mrvs = [bcast(mrv, (t0 % LANES) + t) for t in range(th)]

                def pass2(j, carry):
                    sl = pl.ds(j * LANES, LANES)
                    gv = g_v[sl]
                    be = b_v[sl]
                    cs = [o[t0 + t, sl] for t in range(th)]
                    res = [(cs[t] * rvs[t] - mrvs[t]) * gv + be
                           for t in range(th)]
                    for t in range(th):
                        o[t0 + t, sl] = res[t]
                    return carry

                lax.fori_loop(0, NVREG, pass2, 0, unroll=3)

        def step(k, p):
            wait_gathers(p)
            @pl.when(k + 2 < n_chunks)
            def _():
                issue_ids(k + 2, p)
            q = 1 - p
            @pl.when(k + 1 < n_chunks)
            def _():
                wait_ids(q)
                issue_gathers(k + 1, q)
            @pl.when(k >= 2)
            def _():
                wait_out(p)
            s_local, _, _ = chunk_coords(k)
            s_lv = jnp.full((LANES,), s_local, jnp.int32)
            for g in range(CHUNK // LANES):
                compute_group(p, g, s_lv)
            issue_out(k, p)

        issue_ids(0, 0)
        issue_ids(1, 1)
        wait_ids(0)
        issue_gathers(0, 0)

        def pair_body(gidx, carry):
            step(2 * gidx, 0)
            step(2 * gidx + 1, 1)
            return carry

        lax.fori_loop(0, n_chunks // 2, pair_body, 0)
        wait_out(0)
        wait_out(1)

    return emb_kernel


def kernel(input_ids, token_type_ids, tok_emb, pos_emb, type_emb, gamma, beta):
    try:
        info = plsc.get_sparse_core_info()
        nc, ns = info.num_cores, info.num_subcores
    except Exception:
        nc, ns = 2, 16
    emb_kernel = _build_kernel(nc, ns)
    flat_ids = input_ids.T.reshape(-1)
    flat_tids = token_type_ids.T.reshape(-1)
    out = emb_kernel(flat_ids, flat_tids, _pack_table(tok_emb),
                     _pack_table(pos_emb), _pack_table(type_emb), gamma, beta)
    return out.reshape(B, S, H)


# speedup vs baseline: 1.0326x; 1.0326x over previous
import functools

import jax
import jax.numpy as jnp
from jax import lax
from jax.experimental import pallas as pl
from jax.experimental.pallas import tpu as pltpu
from jax.experimental.pallas import tpu_sc as plsc

B, S, H = 1024, 200, 768
LANES = 16
NVREG = H // LANES
HW = H // 2
CHUNK = 32
NPHASE = 25
EPS = 1e-12


def _rsqrt_vec(v):
    i = lax.bitcast_convert_type(v, jnp.int32)
    i = jnp.int32(0x5F3759DF) - (i >> 1)
    y = lax.bitcast_convert_type(i, jnp.float32)
    half = v * 0.5
    for _ in range(3):
        y = y * (1.5 - half * y * y)
    return y


def _pack_table(x):
    xb = x.astype(jnp.bfloat16)
    u = lax.bitcast_convert_type(xb, jnp.uint16).astype(jnp.uint32)
    packed = (u[:, :HW] << 16) | u[:, HW:]
    return lax.bitcast_convert_type(packed, jnp.int32)


def _build_kernel(num_cores, num_subcores):
    nw = num_cores * num_subcores
    tokens = B * S
    per_w = tokens // nw
    n_chunks = per_w // CHUNK
    per_phase = n_chunks // NPHASE
    mesh = plsc.VectorSubcoreMesh(core_axis_name="c", subcore_axis_name="s")

    @functools.partial(
        pl.kernel,
        mesh=mesh,
        out_type=jax.ShapeDtypeStruct((tokens, H), jnp.float32),
        compiler_params=pltpu.CompilerParams(needs_layout_passes=False,
                                             use_tc_tiling_on_sc=False),
        scratch_types=(
            [pltpu.VMEM((CHUNK,), jnp.int32) for _ in range(2)]
            + [pltpu.VMEM((CHUNK,), jnp.int32) for _ in range(2)]
            + [pltpu.VMEM((CHUNK, HW), jnp.int32) for _ in range(2)]
            + [pltpu.VMEM((CHUNK, HW), jnp.int32) for _ in range(2)]
            + [pltpu.VMEM((CHUNK, HW), jnp.int32)]
            + [pltpu.VMEM((CHUNK, H), jnp.float32) for _ in range(2)]
            + [pltpu.VMEM((H,), jnp.float32) for _ in range(2)]
            + [pltpu.SemaphoreType.DMA for _ in range(10)]
        ),
    )
    def emb_kernel(ids_hbm, tids_hbm, tok_hbm, pos_hbm, typ_hbm, gamma_hbm,
                   beta_hbm, out_hbm,
                   idtok0, idtok1, idtyp0, idtyp1, tokb0, tokb1, typb0, typb1,
                   pos_pin, ob0, ob1, g_v, b_v,
                   s_gt0, s_gt1, s_gy0, s_gy1,
                   s_it0, s_it1, s_iy0, s_iy1, s_o0, s_o1):
        idtok = (idtok0, idtok1)
        idtyp = (idtyp0, idtyp1)
        tokb = (tokb0, tokb1)
        typb = (typb0, typb1)
        ob = (ob0, ob1)
        s_gt = (s_gt0, s_gt1)
        s_gy = (s_gy0, s_gy1)
        s_it = (s_it0, s_it1)
        s_iy = (s_iy0, s_iy1)
        s_o = (s_o0, s_o1)

        wid = lax.axis_index("s") * num_cores + lax.axis_index("c")
        wbase = wid * per_w
        pltpu.sync_copy(gamma_hbm, g_v)
        pltpu.sync_copy(beta_hbm, b_v)
        row_iota = jnp.arange(LANES, dtype=jnp.int32)

        def chunk_of(m):
            return lax.rem(m, per_phase) * NPHASE + m // per_phase

        def issue_ids(m, p):
            base = wbase + chunk_of(m) * CHUNK
            pltpu.async_copy(ids_hbm.at[pl.ds(base, CHUNK)], idtok[p], s_it[p])
            pltpu.async_copy(tids_hbm.at[pl.ds(base, CHUNK)], idtyp[p], s_iy[p])

        def wait_ids(p):
            pltpu.make_async_copy(ids_hbm.at[pl.ds(0, CHUNK)], idtok[p],
                                  s_it[p]).wait()
            pltpu.make_async_copy(tids_hbm.at[pl.ds(0, CHUNK)], idtyp[p],
                                  s_iy[p]).wait()

        def issue_gathers(p):
            pltpu.async_copy(tok_hbm.at[idtok[p]], tokb[p], s_gt[p])
            pltpu.async_copy(typ_hbm.at[idtyp[p]], typb[p], s_gy[p])

        def wait_gathers(p):
            pltpu.make_async_copy(tok_hbm.at[idtok[p]], tokb[p], s_gt[p]).wait()
            pltpu.make_async_copy(typ_hbm.at[idtyp[p]], typb[p], s_gy[p]).wait()

        def wait_out(p):
            pltpu.make_async_copy(ob[p], out_hbm.at[pl.ds(0, CHUNK)],
                                  s_o[p]).wait()

        bcast_dnums = lax.GatherDimensionNumbers(
            offset_dims=(), collapsed_slice_dims=(0,), start_index_map=(0,))

        def bcast(vec, lane):
            idx = jnp.full((LANES, 1), lane, jnp.int32)
            return lax.gather(vec, idx, dimension_numbers=bcast_dnums,
                              slice_sizes=(1,),
                              mode=lax.GatherScatterMode.PROMISE_IN_BOUNDS)

        def unpack(w):
            hi = lax.bitcast_convert_type(w, jnp.float32)
            lo = lax.bitcast_convert_type(w << 16, jnp.float32)
            return hi, lo

        def compute_group(p, g):
            tb, yb, o = tokb[p], typb[p], ob[p]
            rows = row_iota + g * LANES
            nacc = 2
            ph = 4

            def pass1_body(carry, wrap):
                accs = list(carry[:4 * nacc])
                hvs = list(carry[4 * nacc:])
                tws = [plsc.load_gather(tb, [rows, hvs[u]]) for u in range(ph)]
                yws = [plsc.load_gather(yb, [rows, hvs[u]]) for u in range(ph)]
                pws = [plsc.load_gather(pos_pin, [rows, hvs[u]])
                       for u in range(ph)]
                for u in range(ph):
                    thi, tlo = unpack(tws[u])
                    yhi, ylo = unpack(yws[u])
                    phi, plo = unpack(pws[u])
                    chi = (thi + yhi) + phi
                    clo = (tlo + ylo) + plo
                    plsc.store_scatter(o, [rows, hvs[u]], chi)
                    plsc.store_scatter(o, [rows, hvs[u] + HW], clo)
                    a = u % nacc
                    accs[a] = accs[a] + chi
                    accs[nacc + a] = accs[nacc + a] + clo
                    accs[2 * nacc + a] = accs[2 * nacc + a] + chi * chi
                    accs[3 * nacc + a] = accs[3 * nacc + a] + clo * clo
                nxt = []
                for u in range(ph):
                    hv = hvs[u] + ph
                    if wrap:
                        hv = jnp.where(hv >= HW, hv - HW, hv)
                    nxt.append(hv)
                return tuple(accs) + tuple(nxt)

            zero = jnp.zeros((LANES,), jnp.float32)
            hv0 = [row_iota + u for u in range(ph)]
            n_safe = (HW - LANES - ph) // ph
            carry = lax.fori_loop(0, n_safe,
                                  lambda blk, c: pass1_body(c, False),
                                  (zero,) * (4 * nacc) + tuple(hv0))
            carry = lax.fori_loop(n_safe, HW // ph,
                                  lambda blk, c: pass1_body(c, True),
                                  carry)
            s1 = (carry[0] + carry[1]) + (carry[2] + carry[3])
            s2 = (carry[4] + carry[5]) + (carry[6] + carry[7])
            mv = s1 * (1.0 / H)
            var = jnp.maximum(s2 * (1.0 / H) - mv * mv, 0.0)
            rv = _rsqrt_vec(var + EPS)
            mrv = mv * rv

            th = 8
            for t0 in (g * LANES, g * LANES + th):
                rvs = [bcast(rv, (t0 % LANES) + t) for t in range(th)]
                mrvs = [bcast(mrv, (t0 % LANES) + t) for t in range(th)]

                def pass2(j, carry):
                    sl = pl.ds(j * LANES, LANES)
                    gv = g_v[sl]
                    be = b_v[sl]
                    cs = [o[t0 + t, sl] for t in range(th)]
                    res = [(cs[t] * rvs[t] - mrvs[t]) * gv + be
                           for t in range(th)]
                    for t in range(th):
                        o[t0 + t, sl] = res[t]
                    return carry

                lax.fori_loop(0, NVREG, pass2, 0, unroll=3)

        def step(m, p):
            k = chunk_of(m)
            wait_gathers(p)
            @pl.when(m + 2 < n_chunks)
            def _():
                issue_ids(m + 2, p)
            q = 1 - p
            @pl.when(m + 1 < n_chunks)
            def _():
                wait_ids(q)
                issue_gathers(q)
            @pl.when(lax.rem(m, per_phase) == 0)
            def _():
                poff = lax.rem((m // per_phase) * CHUNK, S)
                pltpu.sync_copy(pos_hbm.at[pl.ds(poff, CHUNK)], pos_pin)
            @pl.when(m >= 2)
            def _():
                wait_out(p)
            for g in range(CHUNK // LANES):
                compute_group(p, g)
            pltpu.async_copy(ob[p], out_hbm.at[pl.ds(wbase + k * CHUNK, CHUNK)],
                             s_o[p])

        issue_ids(0, 0)
        issue_ids(1, 1)
        wait_ids(0)
        issue_gathers(0)

        def pair_body(gidx, carry):
            step(2 * gidx, 0)
            step(2 * gidx + 1, 1)
            return carry

        lax.fori_loop(0, n_chunks // 2, pair_body, 0)
        wait_out(0)
        wait_out(1)

    return emb_kernel


def kernel(input_ids, token_type_ids, tok_emb, pos_emb, type_emb, gamma, beta):
    try:
        info = plsc.get_sparse_core_info()
        nc, ns = info.num_cores, info.num_subcores
    except Exception:
        nc, ns = 2, 16
    emb_kernel = _build_kernel(nc, ns)
    flat_ids = input_ids.reshape(-1)
    flat_tids = token_type_ids.reshape(-1)
    ppos = _pack_table(pos_emb)
    ppos = jnp.concatenate([ppos, ppos[:CHUNK]], axis=0)
    out = emb_kernel(flat_ids, flat_tids, _pack_table(tok_emb), ppos,
                     _pack_table(type_emb), gamma, beta)
    return out.reshape(B, S, H)


# speedup vs baseline: 1.0639x; 1.0303x over previous
import functools

import jax
import jax.numpy as jnp
from jax import lax
from jax.experimental import pallas as pl
from jax.experimental.pallas import tpu as pltpu
from jax.experimental.pallas import tpu_sc as plsc

B, S, H = 1024, 200, 768
LANES = 16
NVREG = H // LANES
HW = H // 2
CHUNK = 32
EPS = 1e-12


def _rsqrt_vec(v):
    i = lax.bitcast_convert_type(v, jnp.int32)
    i = jnp.int32(0x5F3759DF) - (i >> 1)
    y = lax.bitcast_convert_type(i, jnp.float32)
    half = v * 0.5
    for _ in range(3):
        y = y * (1.5 - half * y * y)
    return y


def _pack_table(x):
    xb = x.astype(jnp.bfloat16)
    u = lax.bitcast_convert_type(xb, jnp.uint16).astype(jnp.uint32)
    packed = (u[:, :HW] << 16) | u[:, HW:]
    return lax.bitcast_convert_type(packed, jnp.int32)


def _build_kernel(num_cores, num_subcores):
    nw = num_cores * num_subcores
    tokens = B * S
    per_w = tokens // nw
    n_chunks = per_w // CHUNK
    mesh = plsc.VectorSubcoreMesh(core_axis_name="c", subcore_axis_name="s")

    @functools.partial(
        pl.kernel,
        mesh=mesh,
        out_type=jax.ShapeDtypeStruct((tokens, H), jnp.float32),
        compiler_params=pltpu.CompilerParams(needs_layout_passes=False,
                                             use_tc_tiling_on_sc=False),
        scratch_types=(
            [pltpu.VMEM((CHUNK,), jnp.int32) for _ in range(2)]
            + [pltpu.VMEM((CHUNK,), jnp.int32) for _ in range(2)]
            + [pltpu.VMEM((CHUNK, HW), jnp.int32) for _ in range(2)]
            + [pltpu.VMEM((CHUNK, HW), jnp.int32) for _ in range(2)]
            + [pltpu.VMEM((CHUNK, HW), jnp.int32) for _ in range(2)]
            + [pltpu.VMEM((CHUNK, H), jnp.float32) for _ in range(2)]
            + [pltpu.VMEM((H,), jnp.float32) for _ in range(2)]
            + [pltpu.SemaphoreType.DMA for _ in range(12)]
        ),
    )
    def emb_kernel(ids_hbm, tids_hbm, tok_hbm, pos_hbm, typ_hbm, gamma_hbm,
                   beta_hbm, out_hbm,
                   idtok0, idtok1, idtyp0, idtyp1, tokb0, tokb1, typb0, typb1,
                   posb0, posb1, ob0, ob1, g_v, b_v,
                   s_gt0, s_gt1, s_gy0, s_gy1, s_gp0, s_gp1,
                   s_it0, s_it1, s_iy0, s_iy1, s_o0, s_o1):
        idtok = (idtok0, idtok1)
        idtyp = (idtyp0, idtyp1)
        tokb = (tokb0, tokb1)
        typb = (typb0, typb1)
        posb = (posb0, posb1)
        ob = (ob0, ob1)
        s_gt = (s_gt0, s_gt1)
        s_gy = (s_gy0, s_gy1)
        s_gp = (s_gp0, s_gp1)
        s_it = (s_it0, s_it1)
        s_iy = (s_iy0, s_iy1)
        s_o = (s_o0, s_o1)

        wid = lax.axis_index("s") * num_cores + lax.axis_index("c")
        wbase = wid * per_w
        pltpu.sync_copy(gamma_hbm, g_v)
        pltpu.sync_copy(beta_hbm, b_v)
        row_iota = jnp.arange(LANES, dtype=jnp.int32)

        def issue_ids(m, p):
            base = wbase + m * CHUNK
            pltpu.async_copy(ids_hbm.at[pl.ds(base, CHUNK)], idtok[p], s_it[p])
            pltpu.async_copy(tids_hbm.at[pl.ds(base, CHUNK)], idtyp[p], s_iy[p])

        def wait_ids(p):
            pltpu.make_async_copy(ids_hbm.at[pl.ds(0, CHUNK)], idtok[p],
                                  s_it[p]).wait()
            pltpu.make_async_copy(tids_hbm.at[pl.ds(0, CHUNK)], idtyp[p],
                                  s_iy[p]).wait()

        def issue_gathers(k, p):
            pltpu.async_copy(tok_hbm.at[idtok[p]], tokb[p], s_gt[p])
            pltpu.async_copy(typ_hbm.at[idtyp[p]], typb[p], s_gy[p])
            pa = lax.rem(k * CHUNK + row_iota, S)
            pb_ = lax.rem(k * CHUNK + LANES + row_iota, S)
            pltpu.async_copy(pos_hbm.at[pa], posb[p].at[pl.ds(0, LANES)],
                             s_gp[p])
            pltpu.async_copy(pos_hbm.at[pb_], posb[p].at[pl.ds(LANES, LANES)],
                             s_gp[p])

        def wait_gathers(p):
            pltpu.make_async_copy(tok_hbm.at[idtok[p]], tokb[p], s_gt[p]).wait()
            pltpu.make_async_copy(typ_hbm.at[idtyp[p]], typb[p], s_gy[p]).wait()
            pltpu.make_async_copy(tok_hbm.at[idtok[p]], posb[p], s_gp[p]).wait()

        def wait_out(p):
            pltpu.make_async_copy(ob[p], out_hbm.at[pl.ds(0, CHUNK)],
                                  s_o[p]).wait()

        bcast_dnums = lax.GatherDimensionNumbers(
            offset_dims=(), collapsed_slice_dims=(0,), start_index_map=(0,))

        def bcast(vec, lane):
            idx = jnp.full((LANES, 1), lane, jnp.int32)
            return lax.gather(vec, idx, dimension_numbers=bcast_dnums,
                              slice_sizes=(1,),
                              mode=lax.GatherScatterMode.PROMISE_IN_BOUNDS)

        def unpack(w):
            hi = lax.bitcast_convert_type(w, jnp.float32)
            lo = lax.bitcast_convert_type(w << 16, jnp.float32)
            return hi, lo

        def compute_group(p, g):
            tb, yb, pbuf, o = tokb[p], typb[p], posb[p], ob[p]
            rows = row_iota + g * LANES
            nacc = 2
            ph = 4

            def pass1_body(carry, wrap):
                accs = list(carry[:4 * nacc])
                hvs = list(carry[4 * nacc:])
                tws = [plsc.load_gather(tb, [rows, hvs[u]]) for u in range(ph)]
                yws = [plsc.load_gather(yb, [rows, hvs[u]]) for u in range(ph)]
                pws = [plsc.load_gather(pbuf, [rows, hvs[u]])
                       for u in range(ph)]
                for u in range(ph):
                    thi, tlo = unpack(tws[u])
                    yhi, ylo = unpack(yws[u])
                    phi, plo = unpack(pws[u])
                    chi = (thi + yhi) + phi
                    clo = (tlo + ylo) + plo
                    plsc.store_scatter(o, [rows, hvs[u]], chi)
                    plsc.store_scatter(o, [rows, hvs[u] + HW], clo)
                    a = u % nacc
                    accs[a] = accs[a] + chi
                    accs[nacc + a] = accs[nacc + a] + clo
                    accs[2 * nacc + a] = accs[2 * nacc + a] + chi * chi
                    accs[3 * nacc + a] = accs[3 * nacc + a] + clo * clo
                nxt = []
                for u in range(ph):
                    hv = hvs[u] + ph
                    if wrap:
                        hv = jnp.where(hv >= HW, hv - HW, hv)
                    nxt.append(hv)
                return tuple(accs) + tuple(nxt)

            zero = jnp.zeros((LANES,), jnp.float32)
            hv0 = [row_iota + u for u in range(ph)]
            n_safe = (HW - LANES - ph) // ph
            carry = lax.fori_loop(0, n_safe,
                                  lambda blk, c: pass1_body(c, False),
                                  (zero,) * (4 * nacc) + tuple(hv0))
            carry = lax.fori_loop(n_safe, HW // ph,
                                  lambda blk, c: pass1_body(c, True),
                                  carry)
            s1 = (carry[0] + carry[1]) + (carry[2] + carry[3])
            s2 = (carry[4] + carry[5]) + (carry[6] + carry[7])
            mv = s1 * (1.0 / H)
            var = jnp.maximum(s2 * (1.0 / H) - mv * mv, 0.0)
            rv = _rsqrt_vec(var + EPS)
            mrv = mv * rv

            th = 8
            for t0 in (g * LANES, g * LANES + th):
                rvs = [bcast(rv, (t0 % LANES) + t) for t in range(th)]
                mrvs = [bcast(mrv, (t0 % LANES) + t) for t in range(th)]

                def pass2(j, carry):
                    sl = pl.ds(j * LANES, LANES)
                    gv = g_v[sl]
                    be = b_v[sl]
                    cs = [o[t0 + t, sl] for t in range(th)]
                    res = [(cs[t] * rvs[t] - mrvs[t]) * gv + be
                           for t in range(th)]
                    for t in range(th):
                        o[t0 + t, sl] = res[t]
                    return carry

                lax.fori_loop(0, NVREG, pass2, 0, unroll=3)

        def step(k, p):
            wait_gathers(p)
            @pl.when(k + 2 < n_chunks)
            def _():
                issue_ids(k + 2, p)
            q = 1 - p
            @pl.when(k + 1 < n_chunks)
            def _():
                wait_ids(q)
                issue_gathers(k + 1, q)
            @pl.when(k >= 2)
            def _():
                wait_out(p)
            for g in range(CHUNK // LANES):
                compute_group(p, g)
            pltpu.async_copy(ob[p], out_hbm.at[pl.ds(wbase + k * CHUNK, CHUNK)],
                             s_o[p])

        issue_ids(0, 0)
        issue_ids(1, 1)
        wait_ids(0)
        issue_gathers(0, 0)

        def pair_body(gidx, carry):
            step(2 * gidx, 0)
            step(2 * gidx + 1, 1)
            return carry

        lax.fori_loop(0, n_chunks // 2, pair_body, 0)
        wait_out(0)
        wait_out(1)

    return emb_kernel


def kernel(input_ids, token_type_ids, tok_emb, pos_emb, type_emb, gamma, beta):
    try:
        info = plsc.get_sparse_core_info()
        nc, ns = info.num_cores, info.num_subcores
    except Exception:
        nc, ns = 2, 16
    emb_kernel = _build_kernel(nc, ns)
    flat_ids = input_ids.reshape(-1)
    flat_tids = token_type_ids.reshape(-1)
    out = emb_kernel(flat_ids, flat_tids, _pack_table(tok_emb),
                     _pack_table(pos_emb), _pack_table(type_emb), gamma, beta)
    return out.reshape(B, S, H)
